# baseline (speedup 1.0000x reference)
import jax
import jax.numpy as jnp
from jax import lax
from jax.experimental import pallas as pl
from jax.experimental.pallas import tpu as pltpu
from jax.experimental.pallas import tpu_sc as plsc

_B, _T, _D = 4096, 200, 64
_N = _B * _T
_CTX = 200
_NC, _NS = 2, 16
_NW = _NC * _NS
_BPW = _B // _NW
_RPW = _BPW * _T
_U = 2
_ROWS_PER_BODY = 2 * _U
_NBODY = _BPW // _ROWS_PER_BODY
_SEG = (128, 72)


def _emb_body(x_hbm, pi_hbm, val_tab, pos_tab, out_hbm,
              xi_all, pi_all, pos_vt, bufs, sem_gv, sem_ga, sem_st):
    wid = lax.axis_index("s") * _NC + lax.axis_index("c")
    wb = wid * _BPW
    base0 = wid * _RPW

    pltpu.sync_copy(x_hbm.at[pl.ds(base0, _RPW)], xi_all)
    pltpu.sync_copy(pi_hbm.at[pl.ds(base0, _RPW)], pi_all)
    pl.when(lax.axis_index("s") == 0)(lambda: pltpu.sync_copy(pos_tab, pos_vt))
    plsc.subcore_barrier()

    def row(k, h, u):
        return k * _ROWS_PER_BODY + h * _U + u

    def gv(k, h, u):
        o = 0
        for seg in _SEG:
            pltpu.async_copy(
                val_tab.at[xi_all.at[pl.ds(row(k, h, u) * _T + o, seg)]],
                bufs.at[h, u, pl.ds(o, seg)], sem_gv.at[h, u])
            o += seg

    def gv_wait(h, u):
        o = 0
        for seg in _SEG:
            pltpu.make_async_copy(
                val_tab.at[xi_all.at[pl.ds(o, seg)]],
                bufs.at[h, u, pl.ds(o, seg)], sem_gv.at[h, u]).wait()
            o += seg

    def ga(k, h, u):
        o = 0
        for seg in _SEG:
            pltpu.async_copy(
                pos_vt.at[pi_all.at[pl.ds(row(k, h, u) * _T + o, seg)]],
                bufs.at[h, u, pl.ds(o, seg)], sem_ga.at[h, u], add=True)
            o += seg

    def ga_wait(h, u):
        o = 0
        for seg in _SEG:
            pltpu.make_async_copy(
                pos_vt.at[pi_all.at[pl.ds(o, seg)]],
                bufs.at[h, u, pl.ds(o, seg)], sem_ga.at[h, u]).wait()
            o += seg

    def st(k, h, u):
        pltpu.async_copy(
            bufs.at[h, u], out_hbm.at[wb + row(k, h, u)], sem_st.at[h, u])

    def st_wait(h, u):
        pltpu.make_async_copy(
            bufs.at[h, u], out_hbm.at[wb], sem_st.at[h, u]).wait()

    for u in range(_U):
        gv(0, 0, u)

    def body(k, carry):
        for u in range(_U):
            gv_wait(0, u)
            ga(k, 0, u)
        for u in range(_U):
            pl.when(k > 0)(lambda u=u: st_wait(1, u))
            gv(k, 1, u)
        for u in range(_U):
            ga_wait(0, u)
            st(k, 0, u)
        for u in range(_U):
            gv_wait(1, u)
            ga(k, 1, u)
        for u in range(_U):
            st_wait(0, u)
            pl.when(k < _NBODY - 1)(lambda u=u: gv(k + 1, 0, u))
        for u in range(_U):
            ga_wait(1, u)
            st(k, 1, u)
        return carry

    lax.fori_loop(0, _NBODY, body, 0)

    for u in range(_U):
        st_wait(1, u)


@jax.jit
def _emb(xf, pf, value_table, pos_table):
    f = pl.kernel(
        _emb_body,
        out_type=jax.ShapeDtypeStruct((_B, _T, _D), jnp.float32),
        mesh=plsc.VectorSubcoreMesh(
            core_axis_name="c", subcore_axis_name="s",
            num_cores=_NC, num_subcores=_NS),
        scratch_types=[
            pltpu.VMEM((_RPW,), jnp.int32),
            pltpu.VMEM((_RPW,), jnp.int32),
            pltpu.VMEM_SHARED((_CTX, _D), jnp.float32),
            pltpu.VMEM((2, _U, _T, _D), jnp.float32),
            pltpu.SemaphoreType.DMA((2, _U)),
            pltpu.SemaphoreType.DMA((2, _U)),
            pltpu.SemaphoreType.DMA((2, _U)),
        ],
        compiler_params=pltpu.CompilerParams(
            use_tc_tiling_on_sc=False, skip_device_barrier=True),
    )
    return f(xf, pf, value_table, pos_table)


def kernel(x, pos_idx, value_table, pos_table):
    xf = x.reshape(_N)
    pf = pos_idx.reshape(_N)
    return _emb(xf, pf, value_table, pos_table)
